# baseline (speedup 1.0000x reference)
import functools

import jax
import jax.numpy as jnp
from jax import lax
from jax.experimental import pallas as pl
from jax.experimental.pallas import tpu as pltpu
from jax.experimental.pallas import tpu_sc as plsc

_B = 1024
_S = 520
_EMB = 128
_NC, _NS = 2, 16
_NW = _NC * _NS
_BPW = _B // _NW
_C = 104
_CPE = _S // _C
_NCHUNKS = _BPW * _CPE
_GR = 8
_NG = _C // _GR
_NV = _EMB // 16
_RING = 4
_OW = 16


def _sc_logreg(idx_flat, table, wtb):
    mesh = plsc.VectorSubcoreMesh(
        core_axis_name="c", subcore_axis_name="s",
        num_cores=_NC, num_subcores=_NS)

    @functools.partial(
        pl.kernel,
        out_type=jax.ShapeDtypeStruct((_B * _OW,), jnp.float32),
        mesh=mesh,
        scratch_types=[
            pltpu.VMEM((_BPW * _S,), jnp.int32),
            pltpu.VMEM((_RING, _C, _EMB), jnp.float32),
            pltpu.VMEM((_BPW * _OW,), jnp.float32),
            pltpu.VMEM((528,), jnp.float32),
            pltpu.SemaphoreType.DMA,
            pltpu.SemaphoreType.DMA,
            pltpu.SemaphoreType.DMA,
            pltpu.SemaphoreType.DMA,
        ],
    )
    def k(idx_hbm, table_hbm, wtb_hbm, out_hbm, idx_v, rows_v, out_v, w_v,
          sem0, sem1, sem2, sem3):
        wid = lax.axis_index("s") * _NC + lax.axis_index("c")
        base_e = wid * _BPW
        sems = [sem0, sem1, sem2, sem3]

        dn = lax.GatherDimensionNumbers(
            offset_dims=(), collapsed_slice_dims=(0,), start_index_map=(0,))
        bfly_idx = [(lax.iota(jnp.int32, 16) ^ s).reshape(16, 1)
                    for s in (1, 2, 4, 8)]

        def lanesum(v):
            for idx in bfly_idx:
                v = v + lax.gather(
                    v, idx, dn, slice_sizes=(1,),
                    mode=lax.GatherScatterMode.PROMISE_IN_BOUNDS)
            return v

        pltpu.sync_copy(idx_hbm.at[pl.ds(base_e * _S, _BPW * _S)], idx_v)
        pltpu.sync_copy(wtb_hbm, w_v)

        def dma(kc, buf):
            return pltpu.make_async_copy(
                table_hbm.at[idx_v.at[pl.ds(kc * _C, _C)]],
                rows_v.at[buf], sems[buf])

        for p in range(_RING - 1):
            dma(p, p).start()

        iota = lax.iota(jnp.int32, 16)

        def body_k(kc, carry):
            accs, bests, bestn = carry
            par = lax.rem(kc, _RING)
            for b in range(_RING):
                @pl.when(par == b)
                def _():
                    dma(kc, b).wait()

                @pl.when(jnp.logical_and(par == b,
                                         kc + _RING - 1 < _NCHUNKS))
                def _():
                    dma(kc + _RING - 1, (b + _RING - 1) % _RING).start()

            first = lax.rem(kc, _CPE) == 0
            keep = jnp.where(first, 0.0, 1.0)
            accs = tuple(a * keep for a in accs)
            bestn = bestn * keep - (1.0 - keep)

            rv = rows_v.at[par]

            def grp(g, carry):
                accs, bests, bestn = carry
                for r8 in range(_GR):
                    row = g * _GR + r8
                    regs = [rv[row, pl.ds(16 * j, 16)] for j in range(_NV)]
                    sq = regs[0] * regs[0]
                    for j in range(1, _NV):
                        sq = sq + regs[j] * regs[j]
                    nv = lanesum(sq)
                    m = nv > bestn
                    accs = tuple(a + r for a, r in zip(accs, regs))
                    bests = tuple(jnp.where(m, r, bb)
                                  for bb, r in zip(bests, regs))
                    bestn = jnp.maximum(nv, bestn)
                return accs, bests, bestn

            accs, bests, bestn = lax.fori_loop(
                0, _NG, grp, (accs, bests, bestn))

            @pl.when(lax.rem(kc, _CPE) == _CPE - 1)
            def _():
                e = kc // _CPE
                logits = []
                for c in range(2):
                    t = accs[0] * w_v[pl.ds(c * 256, 16)]
                    for j in range(1, _NV):
                        t = t + accs[j] * w_v[pl.ds(c * 256 + 16 * j, 16)]
                    for j in range(_NV):
                        t = t + bests[j] * w_v[pl.ds(c * 256 + 128 + 16 * j,
                                                     16)]
                    logits.append(lanesum(t))
                bv = w_v[pl.ds(512, 16)]
                o = jnp.where(iota == 0, logits[0],
                              jnp.where(iota == 1, logits[1],
                                        jnp.zeros((16,), jnp.float32)))
                out_v[pl.ds(e * _OW, 16)] = o + bv
            return accs, bests, bestn

        zero = jnp.zeros((16,), jnp.float32)
        init = (tuple(zero for _ in range(_NV)),
                tuple(zero for _ in range(_NV)),
                jnp.full((16,), -1.0, jnp.float32))
        lax.fori_loop(0, _NCHUNKS, body_k, init)
        pltpu.sync_copy(out_v, out_hbm.at[pl.ds(base_e * _OW, _BPW * _OW)])

    return k(idx_flat, table, wtb)


def kernel(indices, embedding_matrix, W, b):
    idx_flat = indices.reshape(-1)
    ws = jnp.concatenate([W[:_EMB] * (1.0 / _S), W[_EMB:]], axis=0)
    wtb = jnp.concatenate(
        [ws.T.reshape(-1), b, jnp.zeros((14,), jnp.float32)])
    out = _sc_logreg(idx_flat, embedding_matrix, wtb)
    return out.reshape(_B, _OW)[:, : W.shape[1]]
